# Initial kernel scaffold; baseline (speedup 1.0000x reference)
import jax
import jax.numpy as jnp
from jax.experimental import pallas as pl


def kernel(input, dilation_map):
    raise NotImplementedError("write your pallas kernel here")


# speedup vs baseline: 34.9411x; 34.9411x over previous
import functools

import jax
import jax.numpy as jnp
from jax.experimental import pallas as pl

_K = 3


def _unfold_body(x_ref, d_ref, o_ref, *, cb, h, w):
    d = d_ref[0]
    is0 = (d == 0)
    is1 = (d == 1)
    for c in range(cb):
        v0 = x_ref[0, c, 0:h, 0:w]
        for ki in range(_K):
            for kj in range(_K):
                v1 = x_ref[0, c, ki:ki + h, kj:kj + w]
                v2 = x_ref[0, c, 2 * ki:2 * ki + h, 2 * kj:2 * kj + w]
                o_ref[0, c, ki * _K + kj] = jnp.where(
                    is0, v0, jnp.where(is1, v1, v2))


@jax.jit
def kernel(input, dilation_map):
    B, C, H, W = input.shape
    xp = jnp.pad(input, ((0, 0), (0, 0), (1, 7), (1, _K)))
    dmap = dilation_map[:, 0]

    cb = 4
    out = pl.pallas_call(
        functools.partial(_unfold_body, cb=cb, h=H, w=W),
        grid=(B, C // cb),
        in_specs=[
            pl.BlockSpec((1, cb, H + 8, W + 4), lambda b, c: (b, c, 0, 0)),
            pl.BlockSpec((1, H, W), lambda b, c: (b, 0, 0)),
        ],
        out_specs=pl.BlockSpec((1, cb, _K * _K, H, W), lambda b, c: (b, c, 0, 0, 0)),
        out_shape=jax.ShapeDtypeStruct((B, C, _K * _K, H, W), input.dtype),
    )(xp, dmap)
    return out.reshape(B, C * _K * _K, H * W)


# speedup vs baseline: 35.6163x; 1.0193x over previous
import functools

import jax
import jax.numpy as jnp
from jax.experimental import pallas as pl

_K = 3


def _shifted(x_ref, c, r, s, h, w):
    v = x_ref[0, c, max(r, 0):h + min(r, 0), max(s, 0):w + min(s, 0)]
    if r > 0:
        v = jnp.concatenate([v, jnp.zeros((r, v.shape[1]), v.dtype)], axis=0)
    elif r < 0:
        v = jnp.concatenate([jnp.zeros((-r, v.shape[1]), v.dtype), v], axis=0)
    if s > 0:
        v = jnp.concatenate([v, jnp.zeros((h, s), v.dtype)], axis=1)
    elif s < 0:
        v = jnp.concatenate([jnp.zeros((h, -s), v.dtype), v], axis=1)
    return v


def _unfold_body(x_ref, d_ref, o_ref, *, cb, h, w):
    d = d_ref[0]
    is0 = (d == 0)
    is1 = (d == 1)
    for c in range(cb):
        v0 = _shifted(x_ref, c, -1, -1, h, w)
        for ki in range(_K):
            for kj in range(_K):
                v1 = _shifted(x_ref, c, ki - 1, kj - 1, h, w)
                v2 = _shifted(x_ref, c, 2 * ki - 1, 2 * kj - 1, h, w)
                o_ref[0, c, ki * _K + kj] = jnp.where(
                    is0, v0, jnp.where(is1, v1, v2))


@jax.jit
def kernel(input, dilation_map):
    B, C, H, W = input.shape
    dmap = dilation_map[:, 0]

    cb = 4
    out = pl.pallas_call(
        functools.partial(_unfold_body, cb=cb, h=H, w=W),
        grid=(B, C // cb),
        in_specs=[
            pl.BlockSpec((1, cb, H, W), lambda b, c: (b, c, 0, 0)),
            pl.BlockSpec((1, H, W), lambda b, c: (b, 0, 0)),
        ],
        out_specs=pl.BlockSpec((1, cb, _K * _K, H, W),
                               lambda b, c: (b, c, 0, 0, 0)),
        out_shape=jax.ShapeDtypeStruct((B, C, _K * _K, H, W), input.dtype),
    )(input, dmap)
    return out.reshape(B, C * _K * _K, H * W)


# speedup vs baseline: 38.7167x; 1.0871x over previous
import functools

import jax
import jax.numpy as jnp
from jax.experimental import pallas as pl

_K = 3


def _shift2d(x_ref, c, q, m, u_dim, l_dim):
    v = x_ref[0, c, max(q, 0):u_dim + min(q, 0), max(m, 0):l_dim + min(m, 0)]
    if q > 0:
        v = jnp.concatenate([v, jnp.zeros((q, v.shape[1]), v.dtype)], axis=0)
    elif q < 0:
        v = jnp.concatenate([jnp.zeros((-q, v.shape[1]), v.dtype), v], axis=0)
    if m > 0:
        v = jnp.concatenate([v, jnp.zeros((u_dim, m), v.dtype)], axis=1)
    elif m < 0:
        v = jnp.concatenate([jnp.zeros((u_dim, -m), v.dtype), v], axis=1)
    return v


def _fshift(x_ref, c, k, u_dim, l_dim):
    q, m = divmod(k, l_dim)
    a = _shift2d(x_ref, c, q, m, u_dim, l_dim)
    if m == 0:
        return a
    b = _shift2d(x_ref, c, q + 1, m - l_dim, u_dim, l_dim)
    return a + b


def _unfold_body(x_ref, d_ref, o_ref, *, cb, h, w, u_dim, l_dim):
    d = d_ref[0]
    is0 = (d == 0)
    is1 = (d == 1)
    fi = (jax.lax.broadcasted_iota(jnp.int32, (u_dim, l_dim), 0) * l_dim
          + jax.lax.broadcasted_iota(jnp.int32, (u_dim, l_dim), 1))
    j = fi - (fi // w) * w
    col_ok = {s: (j + s >= 0) & (j + s < w) for s in (-1, 1, 3)}

    def tap_value(c, r, s):
        v = _fshift(x_ref, c, r * w + s, u_dim, l_dim)
        if s in col_ok:
            v = jnp.where(col_ok[s], v, 0.0)
        return v

    zero = jnp.zeros((u_dim, l_dim), jnp.float32)
    for g in range(cb * 9 // 8):
        rows = []
        v0_cache = {}
        for sub in range(8):
            rl = 8 * g + sub
            c, t = rl // 9, rl % 9
            ki, kj = t // _K, t % _K
            if c not in v0_cache:
                v0_cache[c] = tap_value(c, -1, -1)
            v1 = tap_value(c, ki - 1, kj - 1)
            v2 = tap_value(c, 2 * ki - 1, 2 * kj - 1)
            rows.append(jnp.where(is0, v0_cache[c], jnp.where(is1, v1, v2)))
        t8 = jnp.stack(rows, axis=0).reshape(8, h * w)
        o_ref[0, 8 * g:8 * g + 8, :] = t8


@jax.jit
def kernel(input, dilation_map):
    B, C, H, W = input.shape
    F = H * W
    L = 128
    U = F // L
    xf = input.reshape(B, C, U, L)
    df = dilation_map.reshape(B, U, L)

    cb = 8
    out = pl.pallas_call(
        functools.partial(_unfold_body, cb=cb, h=H, w=W, u_dim=U, l_dim=L),
        grid=(B, C // cb),
        in_specs=[
            pl.BlockSpec((1, cb, U, L), lambda b, c: (b, c, 0, 0)),
            pl.BlockSpec((1, U, L), lambda b, c: (b, 0, 0)),
        ],
        out_specs=pl.BlockSpec((1, cb * _K * _K, F), lambda b, c: (b, c, 0)),
        out_shape=jax.ShapeDtypeStruct((B, C * _K * _K, F), input.dtype),
    )(xf, df)
    return out


# speedup vs baseline: 56.0302x; 1.4472x over previous
import functools

import jax
import jax.numpy as jnp
from jax.experimental import pallas as pl

_K = 3


def _unfold_body(x_ref, d_ref, o_ref, *, cb, w, u_dim, l_dim, su):
    nstrips = u_dim // su
    n_groups = cb * _K * _K // 8
    for st in range(nstrips):
        u0 = st * su
        ds = d_ref[0, u0:u0 + su, :]
        is0 = (ds == 0)
        is1 = (ds == 1)
        fi = ((jax.lax.broadcasted_iota(jnp.int32, (su, l_dim), 0) + u0) * l_dim
              + jax.lax.broadcasted_iota(jnp.int32, (su, l_dim), 1))
        j = fi - (fi // w) * w
        col_ok = {s: (j + s >= 0) & (j + s < w) for s in (-1, 1, 3)}

        xs_cache = {}

        def get_xs(c):
            if c not in xs_cache:
                lo, hi = max(u0 - 2, 0), min(u0 + su + 6, u_dim)
                v = x_ref[0, c, lo:hi, :]
                if u0 - 2 < 0:
                    v = jnp.concatenate(
                        [jnp.zeros((2 - u0, l_dim), v.dtype), v], axis=0)
                if u0 + su + 6 > u_dim:
                    v = jnp.concatenate(
                        [v, jnp.zeros((u0 + su + 6 - u_dim, l_dim), v.dtype)],
                        axis=0)
                xs_cache[c] = v
            return xs_cache[c]

        v0_cache = {}

        def tap_value(c, r, s):
            xs = get_xs(c)
            q, m = divmod(r * w + s, l_dim)

            def sh(qq, mm):
                v = xs[2 + qq:2 + qq + su,
                       max(mm, 0):l_dim + min(mm, 0)]
                if mm > 0:
                    v = jnp.concatenate(
                        [v, jnp.zeros((su, mm), v.dtype)], axis=1)
                elif mm < 0:
                    v = jnp.concatenate(
                        [jnp.zeros((su, -mm), v.dtype), v], axis=1)
                return v

            v = sh(q, m) if m == 0 else sh(q, m) + sh(q + 1, m - l_dim)
            if s in col_ok:
                v = jnp.where(col_ok[s], v, 0.0)
            return v

        for g in range(n_groups):
            rows = []
            for sub in range(8):
                rl = 8 * g + sub
                c, t = rl // 9, rl % 9
                ki, kj = t // _K, t % _K
                if c not in v0_cache:
                    v0_cache[c] = tap_value(c, -1, -1)
                v1 = tap_value(c, ki - 1, kj - 1)
                v2 = tap_value(c, 2 * ki - 1, 2 * kj - 1)
                rows.append(
                    jnp.where(is0, v0_cache[c], jnp.where(is1, v1, v2)))
            t8 = jnp.stack(rows, axis=0).reshape(8, su * l_dim)
            o_ref[0, 8 * g:8 * g + 8, u0 * l_dim:(u0 + su) * l_dim] = t8


@jax.jit
def kernel(input, dilation_map):
    B, C, H, W = input.shape
    F = H * W
    L = 128
    U = F // L
    xf = input.reshape(B, C, U, L)
    df = dilation_map.reshape(B, U, L)

    cb = 8
    out = pl.pallas_call(
        functools.partial(_unfold_body, cb=cb, w=W, u_dim=U, l_dim=L, su=56),
        grid=(B, C // cb),
        in_specs=[
            pl.BlockSpec((1, cb, U, L), lambda b, c: (b, c, 0, 0)),
            pl.BlockSpec((1, U, L), lambda b, c: (b, 0, 0)),
        ],
        out_specs=pl.BlockSpec((1, cb * _K * _K, F), lambda b, c: (b, c, 0)),
        out_shape=jax.ShapeDtypeStruct((B, C * _K * _K, F), input.dtype),
    )(xf, df)
    return out
